# baseline (speedup 1.0000x reference)
import functools

import jax
import jax.numpy as jnp
from jax import lax
from jax.experimental import pallas as pl
from jax.experimental.pallas import tpu as pltpu
from jax.experimental.pallas import tpu_sc as plsc

N = 10000
NP = 10240
D = 256
H = 128
E = 160000
CH = 128
NSUB = 16
NCORE = 2
E_PAD = 163840
NCHUNK = E_PAD // CH
CPT = NCHUNK // NSUB
ROWS_PER_TILE = NP // NSUB


def _gin_aggregate(x2, src2d, dst2d):
    mesh = plsc.VectorSubcoreMesh(core_axis_name="c", subcore_axis_name="s")

    @functools.partial(
        pl.kernel,
        out_type=jax.ShapeDtypeStruct((NCORE, NP, H), jnp.float32),
        mesh=mesh,
        scratch_types=[
            pltpu.VMEM_SHARED((NP, H), jnp.float32),
            pltpu.VMEM((CPT, CH), jnp.int32),
            pltpu.VMEM((CH,), jnp.int32),
            pltpu.VMEM((CH,), jnp.int32),
            pltpu.VMEM((CH, H), jnp.float32),
            pltpu.VMEM((CH, H), jnp.float32),
            pltpu.SemaphoreType.DMA,
            pltpu.SemaphoreType.DMA,
            pltpu.SemaphoreType.DMA,
            pltpu.SemaphoreType.DMA,
            pltpu.SemaphoreType.DMA,
            pltpu.SemaphoreType.DMA,
        ],
    )
    def agg(x2_hbm, src_hbm, dst_hbm, out_hbm, acc, sidx, da, db,
            rows_a, rows_b, sem_ga, sem_gb, sem_sa, sem_sb, sem_ia, sem_ib):
        cid = lax.axis_index("c")
        wid = lax.axis_index("s")
        r0 = wid * ROWS_PER_TILE
        c0 = wid * CPT
        half = x2_hbm.at[cid]
        pltpu.sync_copy(src_hbm.at[pl.ds(c0, CPT)], sidx)
        pltpu.sync_copy(half.at[pl.ds(r0, ROWS_PER_TILE)],
                        acc.at[pl.ds(r0, ROWS_PER_TILE)])
        plsc.subcore_barrier()

        def wait_g(rows, sem):
            pltpu.make_async_copy(half.at[sidx.at[0]], rows, sem).wait()

        def wait_s(rows, buf, sem):
            pltpu.make_async_copy(rows, acc.at[buf], sem).wait()

        def wait_i(buf, sem):
            pltpu.make_async_copy(dst_hbm.at[0], buf, sem).wait()

        pltpu.async_copy(dst_hbm.at[c0], da, sem_ia)
        pltpu.async_copy(dst_hbm.at[c0 + 1], db, sem_ib)
        pltpu.async_copy(half.at[sidx.at[0]], rows_a, sem_ga)
        pltpu.async_copy(half.at[sidx.at[1]], rows_b, sem_gb)

        def body(j, carry):
            na = jnp.minimum(2 * j + 2, CPT - 1)
            nb = jnp.minimum(2 * j + 3, CPT - 1)
            wait_g(rows_a, sem_ga)
            wait_i(da, sem_ia)
            pltpu.async_copy(rows_a, acc.at[da], sem_sa, add=True)
            wait_g(rows_b, sem_gb)
            wait_i(db, sem_ib)
            pltpu.async_copy(rows_b, acc.at[db], sem_sb, add=True)
            wait_s(rows_a, da, sem_sa)
            pltpu.async_copy(half.at[sidx.at[na]], rows_a, sem_ga)
            pltpu.async_copy(dst_hbm.at[c0 + na], da, sem_ia)
            wait_s(rows_b, db, sem_sb)
            pltpu.async_copy(half.at[sidx.at[nb]], rows_b, sem_gb)
            pltpu.async_copy(dst_hbm.at[c0 + nb], db, sem_ib)
            return carry

        lax.fori_loop(0, CPT // 2, body, 0)
        wait_g(rows_a, sem_ga)
        wait_g(rows_b, sem_gb)
        wait_i(da, sem_ia)
        wait_i(db, sem_ib)
        plsc.subcore_barrier()
        pltpu.sync_copy(acc.at[pl.ds(r0, ROWS_PER_TILE)],
                        out_hbm.at[cid, pl.ds(r0, ROWS_PER_TILE)])

    return agg(x2, src2d, dst2d)


BR = 1024


def _mlp_body(x2_ref, wa_ref, ba_ref, wb_ref, bb_ref, relu_out):
    xblk = jnp.concatenate([x2_ref[0], x2_ref[1]], axis=1)
    h = jnp.dot(xblk, wa_ref[...], preferred_element_type=jnp.float32)
    h = jnp.maximum(h + ba_ref[...], 0.0)
    h = jnp.dot(h, wb_ref[...], preferred_element_type=jnp.float32)
    h = h + bb_ref[...]
    if relu_out:
        h = jnp.maximum(h, 0.0)
    return h


def _mlp_split(x2, wa, ba, wb, bb):
    def body(x2_ref, wa_ref, ba_ref, wb_ref, bb_ref, o_ref):
        h = _mlp_body(x2_ref, wa_ref, ba_ref, wb_ref, bb_ref, True)
        o_ref[0] = h[:, :H]
        o_ref[1] = h[:, H:]

    return pl.pallas_call(
        body,
        grid=(NP // BR,),
        in_specs=[
            pl.BlockSpec((NCORE, BR, H), lambda i: (0, i, 0)),
            pl.BlockSpec((D, D), lambda i: (0, 0)),
            pl.BlockSpec((1, D), lambda i: (0, 0)),
            pl.BlockSpec((D, D), lambda i: (0, 0)),
            pl.BlockSpec((1, D), lambda i: (0, 0)),
        ],
        out_specs=pl.BlockSpec((NCORE, BR, H), lambda i: (0, i, 0)),
        out_shape=jax.ShapeDtypeStruct((NCORE, NP, H), jnp.float32),
    )(x2, wa, ba.reshape(1, D), wb, bb.reshape(1, D))


def _mlp_full(x2, wa, ba, wb, bb):
    def body(x2_ref, wa_ref, ba_ref, wb_ref, bb_ref, o_ref):
        o_ref[...] = _mlp_body(x2_ref, wa_ref, ba_ref, wb_ref, bb_ref, False)

    return pl.pallas_call(
        body,
        grid=(NP // BR,),
        in_specs=[
            pl.BlockSpec((NCORE, BR, H), lambda i: (0, i, 0)),
            pl.BlockSpec((D, D), lambda i: (0, 0)),
            pl.BlockSpec((1, D), lambda i: (0, 0)),
            pl.BlockSpec((D, D), lambda i: (0, 0)),
            pl.BlockSpec((1, D), lambda i: (0, 0)),
        ],
        out_specs=pl.BlockSpec((BR, D), lambda i: (i, 0)),
        out_shape=jax.ShapeDtypeStruct((NP, D), jnp.float32),
    )(x2, wa, ba.reshape(1, D), wb, bb.reshape(1, D))


def kernel(x, edge_index, W1a, b1a, W1b, b1b, W2a, b2a, W2b, b2b):
    ei = edge_index.astype(jnp.int32)
    pad = E_PAD - E
    src2d = jnp.concatenate(
        [ei[0], jnp.zeros((pad,), jnp.int32)]).reshape(NCHUNK, CH)
    dst2d = jnp.concatenate(
        [ei[1], N + (jnp.arange(pad, dtype=jnp.int32) % 8)]).reshape(NCHUNK, CH)
    x2 = jnp.pad(x, ((0, NP - N), (0, 0))).reshape(NP, NCORE, H).transpose(1, 0, 2)

    g1 = _gin_aggregate(x2, src2d, dst2d)
    h1 = _mlp_split(g1, W1a, b1a, W1b, b1b)
    g2 = _gin_aggregate(h1, src2d, dst2d)
    return _mlp_full(g2, W2a, b2a, W2b, b2b)[:N]


# speedup vs baseline: 1.0437x; 1.0437x over previous
import functools

import jax
import jax.numpy as jnp
from jax import lax
from jax.experimental import pallas as pl
from jax.experimental.pallas import tpu as pltpu
from jax.experimental.pallas import tpu_sc as plsc

N = 10000
NP = 10240
D = 256
H = 128
E = 160000
CH = 128
NSUB = 16
NCORE = 2
E_PAD = 163840
NCHUNK = E_PAD // CH
CPT = NCHUNK // NSUB
ROWS_PER_TILE = NP // NSUB


def _gin_aggregate(x2, src2d, dst2d):
    mesh = plsc.VectorSubcoreMesh(core_axis_name="c", subcore_axis_name="s")

    @functools.partial(
        pl.kernel,
        out_type=jax.ShapeDtypeStruct((NCORE, NP, H), jnp.float32),
        mesh=mesh,
        scratch_types=[
            pltpu.VMEM_SHARED((NP, H), jnp.float32),
            pltpu.VMEM((CPT, CH), jnp.int32),
            pltpu.VMEM((CH,), jnp.int32),
            pltpu.VMEM((CH,), jnp.int32),
            pltpu.VMEM((CH, H), jnp.float32),
            pltpu.VMEM((CH, H), jnp.float32),
            pltpu.SemaphoreType.DMA,
            pltpu.SemaphoreType.DMA,
            pltpu.SemaphoreType.DMA,
            pltpu.SemaphoreType.DMA,
        ],
    )
    def agg(x2_hbm, src_hbm, dst_hbm, out_hbm, acc, sidx, da, db,
            rows_a, rows_b, sem_ga, sem_gb, sem_ia, sem_ib):
        cid = lax.axis_index("c")
        wid = lax.axis_index("s")
        r0 = wid * ROWS_PER_TILE
        c0 = wid * CPT
        half = x2_hbm.at[cid]
        pltpu.sync_copy(src_hbm.at[pl.ds(c0, CPT)], sidx)
        pltpu.sync_copy(half.at[pl.ds(r0, ROWS_PER_TILE)],
                        acc.at[pl.ds(r0, ROWS_PER_TILE)])
        plsc.subcore_barrier()

        def wait_g(rows, sem):
            pltpu.make_async_copy(half.at[sidx.at[0]], rows, sem).wait()

        def wait_i(buf, sem):
            pltpu.make_async_copy(dst_hbm.at[0], buf, sem).wait()

        pltpu.async_copy(dst_hbm.at[c0], da, sem_ia)
        pltpu.async_copy(dst_hbm.at[c0 + 1], db, sem_ib)
        pltpu.async_copy(half.at[sidx.at[0]], rows_a, sem_ga)

        def body(j, carry):
            cb = 2 * j + 1
            na = jnp.minimum(2 * j + 2, CPT - 1)
            nb = jnp.minimum(2 * j + 3, CPT - 1)
            wait_g(rows_a, sem_ga)
            pltpu.async_copy(half.at[sidx.at[cb]], rows_b, sem_gb)
            wait_i(da, sem_ia)
            pltpu.sync_copy(rows_a, acc.at[da], add=True)
            pltpu.async_copy(dst_hbm.at[c0 + na], da, sem_ia)
            wait_g(rows_b, sem_gb)
            pltpu.async_copy(half.at[sidx.at[na]], rows_a, sem_ga)
            wait_i(db, sem_ib)
            pltpu.sync_copy(rows_b, acc.at[db], add=True)
            pltpu.async_copy(dst_hbm.at[c0 + nb], db, sem_ib)
            return carry

        lax.fori_loop(0, CPT // 2, body, 0)
        wait_g(rows_a, sem_ga)
        wait_i(da, sem_ia)
        wait_i(db, sem_ib)
        plsc.subcore_barrier()
        pltpu.sync_copy(acc.at[pl.ds(r0, ROWS_PER_TILE)],
                        out_hbm.at[cid, pl.ds(r0, ROWS_PER_TILE)])

    return agg(x2, src2d, dst2d)


BR = 1024


def _mlp_block(x2_ref, wa_ref, ba_ref, wb_ref, bb_ref, relu_out):
    xblk = jnp.concatenate([x2_ref[0], x2_ref[1]], axis=1)
    h = jnp.dot(xblk, wa_ref[...], preferred_element_type=jnp.float32)
    h = jnp.maximum(h + ba_ref[...], 0.0)
    h = jnp.dot(h, wb_ref[...], preferred_element_type=jnp.float32)
    h = h + bb_ref[...]
    if relu_out:
        h = jnp.maximum(h, 0.0)
    return h


def _mlp_split(x2, wa, ba, wb, bb):
    def body(x2_ref, wa_ref, ba_ref, wb_ref, bb_ref, o_ref):
        h = _mlp_block(x2_ref, wa_ref, ba_ref, wb_ref, bb_ref, True)
        o_ref[0] = h[:, :H]
        o_ref[1] = h[:, H:]

    return pl.pallas_call(
        body,
        grid=(NP // BR,),
        in_specs=[
            pl.BlockSpec((NCORE, BR, H), lambda i: (0, i, 0)),
            pl.BlockSpec((D, D), lambda i: (0, 0)),
            pl.BlockSpec((1, D), lambda i: (0, 0)),
            pl.BlockSpec((D, D), lambda i: (0, 0)),
            pl.BlockSpec((1, D), lambda i: (0, 0)),
        ],
        out_specs=pl.BlockSpec((NCORE, BR, H), lambda i: (0, i, 0)),
        out_shape=jax.ShapeDtypeStruct((NCORE, NP, H), jnp.float32),
    )(x2, wa, ba.reshape(1, D), wb, bb.reshape(1, D))


BRF = 1000


def _mlp_full(x2, wa, ba, wb, bb):
    def body(x2_ref, wa_ref, ba_ref, wb_ref, bb_ref, o_ref):
        o_ref[...] = _mlp_block(x2_ref, wa_ref, ba_ref, wb_ref, bb_ref, False)

    return pl.pallas_call(
        body,
        grid=(N // BRF,),
        in_specs=[
            pl.BlockSpec((NCORE, BRF, H), lambda i: (0, i, 0)),
            pl.BlockSpec((D, D), lambda i: (0, 0)),
            pl.BlockSpec((1, D), lambda i: (0, 0)),
            pl.BlockSpec((D, D), lambda i: (0, 0)),
            pl.BlockSpec((1, D), lambda i: (0, 0)),
        ],
        out_specs=pl.BlockSpec((BRF, D), lambda i: (i, 0)),
        out_shape=jax.ShapeDtypeStruct((N, D), jnp.float32),
    )(x2, wa, ba.reshape(1, D), wb, bb.reshape(1, D))


def kernel(x, edge_index, W1a, b1a, W1b, b1b, W2a, b2a, W2b, b2b):
    ei = edge_index.astype(jnp.int32)
    pad = E_PAD - E
    src2d = jnp.concatenate(
        [ei[0], jnp.zeros((pad,), jnp.int32)]).reshape(NCHUNK, CH)
    dst2d = jnp.concatenate(
        [ei[1], N + (jnp.arange(pad, dtype=jnp.int32) % 8)]).reshape(NCHUNK, CH)
    x2 = jnp.pad(x, ((0, NP - N), (0, 0))).reshape(NP, NCORE, H).transpose(1, 0, 2)

    g1 = _gin_aggregate(x2, src2d, dst2d)
    h1 = _mlp_split(g1, W1a, b1a, W1b, b1b)
    g2 = _gin_aggregate(h1, src2d, dst2d)
    return _mlp_full(g2, W2a, b2a, W2b, b2b)
